# Initial kernel scaffold; baseline (speedup 1.0000x reference)
import jax
import jax.numpy as jnp
from jax.experimental import pallas as pl


def kernel(x_source, neighborhood, W1, W2, W3):
    raise NotImplementedError("write your pallas kernel here")


# speedup vs baseline: 1.1379x; 1.1379x over previous
import jax
import jax.numpy as jnp
from jax.experimental import pallas as pl


def _proj_kernel(x_ref, w_ref, t_ref):
    t = jax.lax.dot_general(
        x_ref[...], w_ref[...], (((1,), (0,)), ((), ())),
        preferred_element_type=jnp.float32,
        precision=jax.lax.Precision.HIGHEST)
    t_ref[...] = t.astype(jnp.bfloat16)


def _spmm_relu_kernel(a_ref, t_ref, o_ref):
    a = a_ref[...].astype(jnp.bfloat16)
    acc = jax.lax.dot_general(
        a, t_ref[...], (((1,), (0,)), ((), ())),
        preferred_element_type=jnp.float32)
    o_ref[...] = jnp.maximum(acc, 0.0)


def kernel(x_source, neighborhood, W1, W2, W3):
    n, d_in = x_source.shape
    d_out = W1.shape[1]
    bt = min(1024, n)
    bi = min(256, n)

    t = pl.pallas_call(
        _proj_kernel,
        grid=(n // bt,),
        in_specs=[pl.BlockSpec((bt, d_in), lambda i: (i, 0)),
                  pl.BlockSpec((d_in, d_out), lambda i: (0, 0))],
        out_specs=pl.BlockSpec((bt, d_out), lambda i: (i, 0)),
        out_shape=jax.ShapeDtypeStruct((n, d_out), jnp.bfloat16),
    )(x_source, W1)

    out = pl.pallas_call(
        _spmm_relu_kernel,
        grid=(n // bi,),
        in_specs=[pl.BlockSpec((bi, n), lambda i: (i, 0)),
                  pl.BlockSpec((n, d_out), lambda i: (0, 0))],
        out_specs=pl.BlockSpec((bi, d_out), lambda i: (i, 0)),
        out_shape=jax.ShapeDtypeStruct((n, d_out), jnp.float32),
    )(neighborhood, t)
    return out


# speedup vs baseline: 1.2179x; 1.0703x over previous
import jax
import jax.numpy as jnp
from jax.experimental import pallas as pl


def _proj_kernel(x_ref, w_ref, t_ref):
    t = jax.lax.dot_general(
        x_ref[...], w_ref[...], (((1,), (0,)), ((), ())),
        preferred_element_type=jnp.float32,
        precision=jax.lax.Precision.HIGHEST)
    t_ref[...] = t.astype(jnp.bfloat16)


def _spmm_relu_kernel(a_ref, t_ref, o_ref):
    a = a_ref[...].astype(jnp.bfloat16)
    acc = jax.lax.dot_general(
        a, t_ref[...], (((1,), (0,)), ((), ())),
        preferred_element_type=jnp.float32)
    o_ref[...] = jnp.maximum(acc, 0.0)


def kernel(x_source, neighborhood, W1, W2, W3):
    n, d_in = x_source.shape
    d_out = W1.shape[1]
    bt = min(1024, n)
    bi = min(512, n)

    t = pl.pallas_call(
        _proj_kernel,
        grid=(n // bt,),
        in_specs=[pl.BlockSpec((bt, d_in), lambda i: (i, 0)),
                  pl.BlockSpec((d_in, d_out), lambda i: (0, 0))],
        out_specs=pl.BlockSpec((bt, d_out), lambda i: (i, 0)),
        out_shape=jax.ShapeDtypeStruct((n, d_out), jnp.bfloat16),
    )(x_source, W1)

    out = pl.pallas_call(
        _spmm_relu_kernel,
        grid=(n // bi,),
        in_specs=[pl.BlockSpec((bi, n), lambda i: (i, 0)),
                  pl.BlockSpec((n, d_out), lambda i: (0, 0))],
        out_specs=pl.BlockSpec((bi, d_out), lambda i: (i, 0)),
        out_shape=jax.ShapeDtypeStruct((n, d_out), jnp.float32),
    )(neighborhood, t)
    return out


# speedup vs baseline: 1.4204x; 1.1663x over previous
import jax
import jax.numpy as jnp
from jax.experimental import pallas as pl


def _proj_kernel(x_ref, w_ref, t_ref):
    t = jax.lax.dot_general(
        x_ref[...].astype(jnp.bfloat16), w_ref[...].astype(jnp.bfloat16),
        (((1,), (0,)), ((), ())),
        preferred_element_type=jnp.float32)
    t_ref[...] = t.astype(jnp.bfloat16)


def _spmm_relu_kernel(a_ref, t_ref, o_ref):
    a = a_ref[...].astype(jnp.bfloat16)
    acc = jax.lax.dot_general(
        a, t_ref[...], (((1,), (0,)), ((), ())),
        preferred_element_type=jnp.float32)
    o_ref[...] = jnp.maximum(acc, 0.0)


def kernel(x_source, neighborhood, W1, W2, W3):
    n, d_in = x_source.shape
    d_out = W1.shape[1]
    bt = min(1024, n)
    bi = min(512, n)

    t = pl.pallas_call(
        _proj_kernel,
        grid=(n // bt,),
        in_specs=[pl.BlockSpec((bt, d_in), lambda i: (i, 0)),
                  pl.BlockSpec((d_in, d_out), lambda i: (0, 0))],
        out_specs=pl.BlockSpec((bt, d_out), lambda i: (i, 0)),
        out_shape=jax.ShapeDtypeStruct((n, d_out), jnp.bfloat16),
    )(x_source, W1)

    out = pl.pallas_call(
        _spmm_relu_kernel,
        grid=(n // bi,),
        in_specs=[pl.BlockSpec((bi, n), lambda i: (i, 0)),
                  pl.BlockSpec((n, d_out), lambda i: (0, 0))],
        out_specs=pl.BlockSpec((bi, d_out), lambda i: (i, 0)),
        out_shape=jax.ShapeDtypeStruct((n, d_out), jnp.float32),
    )(neighborhood, t)
    return out
